# baseline (speedup 1.0000x reference)
import functools

import jax
import jax.numpy as jnp
from jax import lax
from jax.experimental import pallas as pl
from jax.experimental.pallas import tpu as pltpu
from jax.experimental.pallas import tpu_sc as plsc

B = 16384
NC, NS = 2, 16
NW = NC * NS
BPW = B // NW
CHUNK = 128
HALF = 256
NBUF = 3

ORIG_D = [64, 64, 32, 32, 16, 16, 16, 32, 32]
STD_D = [32, 64, 64, 32, 32, 16, 16, 16, 32, 32]
ALL_D = ORIG_D + STD_D
NT = 19


def _offsets(ds):
    offs, c = [], 0
    for d in ds:
        offs.append(c)
        c += d
    return offs

ORIG_OFF = _offsets(ORIG_D)
STD_OFF = _offsets(STD_D)
D_ORIG_TOT = sum(ORIG_D)
D_STD_TOT = sum(STD_D)

TBL_OUT = [(0, ORIG_OFF[t]) for t in range(9)] + [(1, STD_OFF[t]) for t in range(10)]
UNITS = [(t, h) for t in range(NT) for h in range(BPW // HALF)]


def _body(*refs):
    idx = list(refs[0:NT])
    tabs = list(refs[NT:2 * NT])
    outs = [refs[2 * NT], refs[2 * NT + 1]]
    o = 2 * NT + 2
    iv = refs[o]
    rows = [refs[o + 1], refs[o + 2], refs[o + 3]]
    gsems = [refs[o + 4], refs[o + 5], refs[o + 6]]
    osems = [refs[o + 7], refs[o + 8], refs[o + 9]]
    isem = refs[o + 10]

    wid = lax.axis_index("s") * NC + lax.axis_index("c")
    base = wid * BPW

    idescs = [
        pltpu.async_copy(idx[t].at[pl.ds(base, BPW)],
                         iv.at[pl.ds(t * BPW, BPW)], isem)
        for t in range(NT)
    ]
    for dsc in idescs:
        dsc.wait()

    def out_slice(u):
        t, h = UNITS[u]
        oi, c0 = TBL_OUT[t]
        d = ALL_D[t]
        return outs[oi].at[pl.ds(base + h * HALF, HALF), pl.ds(c0, d)]

    def fire(u):
        t, h = UNITS[u]
        s = u % NBUF
        if u >= NBUF:
            pltpu.make_async_copy(
                rows[s].at[:, pl.ds(0, ALL_D[UNITS[u - NBUF][0]])],
                out_slice(u - NBUF), osems[s],
            ).wait()
        return [
            pltpu.async_copy(
                tabs[t].at[iv.at[pl.ds(t * BPW + h * HALF + j * CHUNK, CHUNK)]],
                rows[s].at[pl.ds(j * CHUNK, CHUNK)],
                gsems[s],
            )
            for j in range(HALF // CHUNK)
        ]

    def finish(u, descs):
        t, _h = UNITS[u]
        s = u % NBUF
        d = ALL_D[t]
        for dsc in descs:
            dsc.wait()
        pltpu.async_copy(rows[s].at[:, pl.ds(0, d)], out_slice(u), osems[s])

    nu = len(UNITS)
    pend = [fire(u) for u in range(NBUF - 1)]
    for u in range(nu):
        if u + NBUF - 1 < nu:
            pend.append(fire(u + NBUF - 1))
        finish(u, pend[u])
    for u in range(nu - NBUF, nu):
        s = u % NBUF
        pltpu.make_async_copy(
            rows[s].at[:, pl.ds(0, ALL_D[UNITS[u][0]])],
            out_slice(u), osems[s],
        ).wait()


@jax.jit
def _run(idxs, tabs):
    mesh = plsc.VectorSubcoreMesh(
        core_axis_name="c", subcore_axis_name="s", num_cores=NC, num_subcores=NS
    )
    fn = pl.kernel(
        _body,
        out_type=(
            jax.ShapeDtypeStruct((B, D_ORIG_TOT), jnp.float32),
            jax.ShapeDtypeStruct((B, D_STD_TOT), jnp.float32),
        ),
        mesh=mesh,
        scratch_types=(
            pltpu.VMEM((NT * BPW,), jnp.int32),
            pltpu.VMEM((HALF, 128), jnp.float32),
            pltpu.VMEM((HALF, 128), jnp.float32),
            pltpu.VMEM((HALF, 128), jnp.float32),
        ) + (pltpu.SemaphoreType.DMA,) * 7,
        compiler_params=pltpu.CompilerParams(use_tc_tiling_on_sc=False),
    )
    return fn(*idxs, *tabs)


def kernel(contact_idx, W_orig_contact, bodypart_idx, W_orig_bodypart, upper_bodypart_idx, W_orig_upper_bodypart, lower_bodypart_idx, W_orig_lower_bodypart, multiple_fouls_idx, W_orig_multiple_fouls, try_to_play_idx, W_orig_try_to_play, touch_ball_idx, W_orig_touch_ball, handball_idx, W_orig_handball, handball_offence_idx, W_orig_handball_offence, offence_standard_idx, W_std_offence, contact_standard_idx, W_std_contact, bodypart_standard_idx, W_std_bodypart, upper_bodypart_standard_idx, W_std_upper_bodypart, lower_bodypart_standard_idx, W_std_lower_bodypart, multiple_fouls_standard_idx, W_std_multiple_fouls, try_to_play_standard_idx, W_std_try_to_play, touch_ball_standard_idx, W_std_touch_ball, handball_standard_idx, W_std_handball, handball_offence_standard_idx, W_std_handball_offence):
    idxs = [contact_idx, bodypart_idx, upper_bodypart_idx, lower_bodypart_idx,
            multiple_fouls_idx, try_to_play_idx, touch_ball_idx, handball_idx,
            handball_offence_idx,
            offence_standard_idx, contact_standard_idx, bodypart_standard_idx,
            upper_bodypart_standard_idx, lower_bodypart_standard_idx,
            multiple_fouls_standard_idx, try_to_play_standard_idx,
            touch_ball_standard_idx, handball_standard_idx,
            handball_offence_standard_idx]
    tabs = [W_orig_contact, W_orig_bodypart, W_orig_upper_bodypart,
            W_orig_lower_bodypart, W_orig_multiple_fouls, W_orig_try_to_play,
            W_orig_touch_ball, W_orig_handball, W_orig_handball_offence,
            W_std_offence, W_std_contact, W_std_bodypart, W_std_upper_bodypart,
            W_std_lower_bodypart, W_std_multiple_fouls, W_std_try_to_play,
            W_std_touch_ball, W_std_handball, W_std_handball_offence]
    tabs = [jnp.pad(w, ((0, 0), (0, 128 - w.shape[1]))) for w in tabs]
    return _run(idxs, tabs)


# speedup vs baseline: 1.0951x; 1.0951x over previous
import functools

import jax
import jax.numpy as jnp
from jax import lax
from jax.experimental import pallas as pl
from jax.experimental.pallas import tpu as pltpu
from jax.experimental.pallas import tpu_sc as plsc

B = 16384
NC, NS = 2, 16
NW = NC * NS
BPW = B // NW
CHUNK = 128
NCHUNK = BPW // CHUNK

ORIG_D = [64, 64, 32, 32, 16, 16, 16, 32, 32]
STD_D = [32, 64, 64, 32, 32, 16, 16, 16, 32, 32]
ALL_D = ORIG_D + STD_D
NT = 19


def _offsets(ds):
    offs, c = [], 0
    for d in ds:
        offs.append(c)
        c += d
    return offs

ORIG_OFF = _offsets(ORIG_D)
STD_OFF = _offsets(STD_D)
D_ORIG_TOT = sum(ORIG_D)
D_STD_TOT = sum(STD_D)

TBL_OUT = [(0, ORIG_OFF[t]) for t in range(9)] + [(1, STD_OFF[t]) for t in range(10)]
CLS = {64: 0, 32: 1, 16: 2}
_by_cls = {0: [], 1: [], 2: []}
for _t in range(NT):
    _by_cls[CLS[ALL_D[_t]]].append(_t)
ORDER = []
_i = 0
while any(_by_cls.values()):
    c = _i % 3
    if _by_cls[c]:
        ORDER.append(_by_cls[c].pop(0))
    _i += 1


def _body(*refs):
    idx = list(refs[0:NT])
    tabs = list(refs[NT:2 * NT])
    outs = [refs[2 * NT], refs[2 * NT + 1]]
    o = 2 * NT + 2
    iv = refs[o]
    rows = {
        0: [refs[o + 1], refs[o + 2]],
        1: [refs[o + 3], refs[o + 4]],
        2: [refs[o + 5], refs[o + 6]],
    }
    gsems = {0: [refs[o + 7], refs[o + 8]],
             1: [refs[o + 9], refs[o + 10]],
             2: [refs[o + 11], refs[o + 12]]}
    osems = {0: [refs[o + 13], refs[o + 14]],
             1: [refs[o + 15], refs[o + 16]],
             2: [refs[o + 17], refs[o + 18]]}
    isem = refs[o + 19]

    wid = lax.axis_index("s") * NC + lax.axis_index("c")
    base = wid * BPW

    idescs = {}
    for t in ORDER:
        idescs[t] = pltpu.async_copy(
            idx[t].at[pl.ds(base, BPW)], iv.at[pl.ds(t * BPW, BPW)], isem)

    def src(t):
        return tabs[t]

    cls_count = {0: 0, 1: 0, 2: 0}
    state = []

    def fire(t):
        d = ALL_D[t]
        c = CLS[d]
        p = cls_count[c] % 2
        if cls_count[c] >= 2:
            oi, c0 = TBL_OUT[t]
            pltpu.make_async_copy(
                rows[c][p],
                outs[oi].at[pl.ds(base, BPW), pl.ds(c0, d)],
                osems[c][p],
            ).wait()
        cls_count[c] += 1
        idescs[t].wait()
        descs = [
            pltpu.async_copy(
                src(t).at[iv.at[pl.ds(t * BPW + j * CHUNK, CHUNK)]],
                rows[c][p].at[pl.ds(j * CHUNK, CHUNK)],
                gsems[c][p],
            )
            for j in range(NCHUNK)
        ]
        state.append((t, c, p, descs))

    def finish(t, c, p, descs):
        for dsc in descs:
            dsc.wait()
        oi, c0 = TBL_OUT[t]
        d = ALL_D[t]
        pltpu.async_copy(
            rows[c][p],
            outs[oi].at[pl.ds(base, BPW), pl.ds(c0, d)],
            osems[c][p],
        )

    fire(ORDER[0])
    for i in range(NT):
        if i + 1 < NT:
            fire(ORDER[i + 1])
        finish(*state[i])
    last = {}
    for (t, c, p, _descs) in state:
        last[(c, p)] = t
    for (c, p), t in last.items():
        oi, c0 = TBL_OUT[t]
        d = ALL_D[t]
        pltpu.make_async_copy(
            rows[c][p],
            outs[oi].at[pl.ds(base, BPW), pl.ds(c0, d)],
            osems[c][p],
        ).wait()


@jax.jit
def _run(idxs, tabs):
    mesh = plsc.VectorSubcoreMesh(
        core_axis_name="c", subcore_axis_name="s", num_cores=NC, num_subcores=NS
    )
    fn = pl.kernel(
        _body,
        out_type=(
            jax.ShapeDtypeStruct((B, D_ORIG_TOT), jnp.float32),
            jax.ShapeDtypeStruct((B, D_STD_TOT), jnp.float32),
        ),
        mesh=mesh,
        scratch_types=(
            pltpu.VMEM((NT * BPW,), jnp.int32),
            pltpu.VMEM((BPW, 64), jnp.float32),
            pltpu.VMEM((BPW, 64), jnp.float32),
            pltpu.VMEM((BPW, 32), jnp.float32),
            pltpu.VMEM((BPW, 32), jnp.float32),
            pltpu.VMEM((BPW, 16), jnp.float32),
            pltpu.VMEM((BPW, 16), jnp.float32),
        ) + (pltpu.SemaphoreType.DMA,) * 13,
        compiler_params=pltpu.CompilerParams(use_tc_tiling_on_sc=False),
    )
    return fn(*idxs, *tabs)


def kernel(contact_idx, W_orig_contact, bodypart_idx, W_orig_bodypart, upper_bodypart_idx, W_orig_upper_bodypart, lower_bodypart_idx, W_orig_lower_bodypart, multiple_fouls_idx, W_orig_multiple_fouls, try_to_play_idx, W_orig_try_to_play, touch_ball_idx, W_orig_touch_ball, handball_idx, W_orig_handball, handball_offence_idx, W_orig_handball_offence, offence_standard_idx, W_std_offence, contact_standard_idx, W_std_contact, bodypart_standard_idx, W_std_bodypart, upper_bodypart_standard_idx, W_std_upper_bodypart, lower_bodypart_standard_idx, W_std_lower_bodypart, multiple_fouls_standard_idx, W_std_multiple_fouls, try_to_play_standard_idx, W_std_try_to_play, touch_ball_standard_idx, W_std_touch_ball, handball_standard_idx, W_std_handball, handball_offence_standard_idx, W_std_handball_offence):
    idxs = [contact_idx, bodypart_idx, upper_bodypart_idx, lower_bodypart_idx,
            multiple_fouls_idx, try_to_play_idx, touch_ball_idx, handball_idx,
            handball_offence_idx,
            offence_standard_idx, contact_standard_idx, bodypart_standard_idx,
            upper_bodypart_standard_idx, lower_bodypart_standard_idx,
            multiple_fouls_standard_idx, try_to_play_standard_idx,
            touch_ball_standard_idx, handball_standard_idx,
            handball_offence_standard_idx]
    tabs = [W_orig_contact, W_orig_bodypart, W_orig_upper_bodypart,
            W_orig_lower_bodypart, W_orig_multiple_fouls, W_orig_try_to_play,
            W_orig_touch_ball, W_orig_handball, W_orig_handball_offence,
            W_std_offence, W_std_contact, W_std_bodypart, W_std_upper_bodypart,
            W_std_lower_bodypart, W_std_multiple_fouls, W_std_try_to_play,
            W_std_touch_ball, W_std_handball, W_std_handball_offence]
    return _run(idxs, tabs)


# speedup vs baseline: 1.0962x; 1.0010x over previous
import functools

import jax
import jax.numpy as jnp
from jax import lax
from jax.experimental import pallas as pl
from jax.experimental.pallas import tpu as pltpu
from jax.experimental.pallas import tpu_sc as plsc

B = 16384
NC, NS = 2, 16
NW = NC * NS
BPW = B // NW
CHUNK = 128
NCHUNK = BPW // CHUNK

ORIG_D = [64, 64, 32, 32, 16, 16, 16, 32, 32]
STD_D = [32, 64, 64, 32, 32, 16, 16, 16, 32, 32]
NT = 19
PERM = [4, 5, 6, 2, 3, 7, 8, 0, 1] + list(range(9, 19))
_ALL_D0 = ORIG_D + STD_D
ALL_D = [_ALL_D0[t] for t in PERM]


def _offsets(ds):
    offs, c = [], 0
    for d in ds:
        offs.append(c)
        c += d
    return offs

ORIG_OFF = _offsets(ORIG_D)
STD_OFF = _offsets(STD_D)
D_ORIG_TOT = sum(ORIG_D)
D_STD_TOT = sum(STD_D)

_TBL_OUT0 = [(0, ORIG_OFF[t]) for t in range(9)] + [(1, STD_OFF[t]) for t in range(10)]
TBL_OUT = [_TBL_OUT0[t] for t in PERM]
CLS = {64: 0, 32: 1, 16: 2}
_by_cls = {0: [], 1: [], 2: []}
for _t in range(NT):
    _by_cls[CLS[ALL_D[_t]]].append(_t)
ORDER = []
_i = 0
while any(_by_cls.values()):
    c = _i % 3
    if _by_cls[c]:
        ORDER.append(_by_cls[c].pop(0))
    _i += 1


def _body(*refs):
    idx = list(refs[0:NT])
    tabs = list(refs[NT:2 * NT])
    outs = [refs[2 * NT], refs[2 * NT + 1]]
    o = 2 * NT + 2
    iv = refs[o]
    rows = {
        0: [refs[o + 1], refs[o + 2]],
        1: [refs[o + 3], refs[o + 4]],
        2: [refs[o + 5], refs[o + 6]],
    }
    gsems = {0: [refs[o + 7], refs[o + 8]],
             1: [refs[o + 9], refs[o + 10]],
             2: [refs[o + 11], refs[o + 12]]}
    osems = {0: [refs[o + 13], refs[o + 14]],
             1: [refs[o + 15], refs[o + 16]],
             2: [refs[o + 17], refs[o + 18]]}
    isem = refs[o + 19]

    wid = lax.axis_index("s") * NC + lax.axis_index("c")
    base = wid * BPW

    idescs = {}
    for t in ORDER:
        idescs[t] = pltpu.async_copy(
            idx[t].at[pl.ds(base, BPW)], iv.at[pl.ds(t * BPW, BPW)], isem)

    def src(t):
        return tabs[t]

    cls_count = {0: 0, 1: 0, 2: 0}
    state = []

    def fire(t):
        d = ALL_D[t]
        c = CLS[d]
        p = cls_count[c] % 2
        if cls_count[c] >= 2:
            oi, c0 = TBL_OUT[t]
            pltpu.make_async_copy(
                rows[c][p],
                outs[oi].at[pl.ds(base, BPW), pl.ds(c0, d)],
                osems[c][p],
            ).wait()
        cls_count[c] += 1
        idescs[t].wait()
        descs = [
            pltpu.async_copy(
                src(t).at[iv.at[pl.ds(t * BPW + j * CHUNK, CHUNK)]],
                rows[c][p].at[pl.ds(j * CHUNK, CHUNK)],
                gsems[c][p],
            )
            for j in range(NCHUNK)
        ]
        state.append((t, c, p, descs))

    def finish(t, c, p, descs):
        for dsc in descs:
            dsc.wait()
        oi, c0 = TBL_OUT[t]
        d = ALL_D[t]
        pltpu.async_copy(
            rows[c][p],
            outs[oi].at[pl.ds(base, BPW), pl.ds(c0, d)],
            osems[c][p],
        )

    fire(ORDER[0])
    for i in range(NT):
        if i + 1 < NT:
            fire(ORDER[i + 1])
        finish(*state[i])
    last = {}
    for (t, c, p, _descs) in state:
        last[(c, p)] = t
    for (c, p), t in last.items():
        oi, c0 = TBL_OUT[t]
        d = ALL_D[t]
        pltpu.make_async_copy(
            rows[c][p],
            outs[oi].at[pl.ds(base, BPW), pl.ds(c0, d)],
            osems[c][p],
        ).wait()


@jax.jit
def _run(idxs, tabs):
    mesh = plsc.VectorSubcoreMesh(
        core_axis_name="c", subcore_axis_name="s", num_cores=NC, num_subcores=NS
    )
    fn = pl.kernel(
        _body,
        out_type=(
            jax.ShapeDtypeStruct((B, D_ORIG_TOT), jnp.float32),
            jax.ShapeDtypeStruct((B, D_STD_TOT), jnp.float32),
        ),
        mesh=mesh,
        scratch_types=(
            pltpu.VMEM((NT * BPW,), jnp.int32),
            pltpu.VMEM((BPW, 64), jnp.float32),
            pltpu.VMEM((BPW, 64), jnp.float32),
            pltpu.VMEM((BPW, 32), jnp.float32),
            pltpu.VMEM((BPW, 32), jnp.float32),
            pltpu.VMEM((BPW, 16), jnp.float32),
            pltpu.VMEM((BPW, 16), jnp.float32),
        ) + (pltpu.SemaphoreType.DMA,) * 13,
        compiler_params=pltpu.CompilerParams(use_tc_tiling_on_sc=False),
    )
    return fn(*idxs, *tabs)


def kernel(contact_idx, W_orig_contact, bodypart_idx, W_orig_bodypart, upper_bodypart_idx, W_orig_upper_bodypart, lower_bodypart_idx, W_orig_lower_bodypart, multiple_fouls_idx, W_orig_multiple_fouls, try_to_play_idx, W_orig_try_to_play, touch_ball_idx, W_orig_touch_ball, handball_idx, W_orig_handball, handball_offence_idx, W_orig_handball_offence, offence_standard_idx, W_std_offence, contact_standard_idx, W_std_contact, bodypart_standard_idx, W_std_bodypart, upper_bodypart_standard_idx, W_std_upper_bodypart, lower_bodypart_standard_idx, W_std_lower_bodypart, multiple_fouls_standard_idx, W_std_multiple_fouls, try_to_play_standard_idx, W_std_try_to_play, touch_ball_standard_idx, W_std_touch_ball, handball_standard_idx, W_std_handball, handball_offence_standard_idx, W_std_handball_offence):
    idxs = [contact_idx, bodypart_idx, upper_bodypart_idx, lower_bodypart_idx,
            multiple_fouls_idx, try_to_play_idx, touch_ball_idx, handball_idx,
            handball_offence_idx,
            offence_standard_idx, contact_standard_idx, bodypart_standard_idx,
            upper_bodypart_standard_idx, lower_bodypart_standard_idx,
            multiple_fouls_standard_idx, try_to_play_standard_idx,
            touch_ball_standard_idx, handball_standard_idx,
            handball_offence_standard_idx]
    tabs = [W_orig_contact, W_orig_bodypart, W_orig_upper_bodypart,
            W_orig_lower_bodypart, W_orig_multiple_fouls, W_orig_try_to_play,
            W_orig_touch_ball, W_orig_handball, W_orig_handball_offence,
            W_std_offence, W_std_contact, W_std_bodypart, W_std_upper_bodypart,
            W_std_lower_bodypart, W_std_multiple_fouls, W_std_try_to_play,
            W_std_touch_ball, W_std_handball, W_std_handball_offence]
    return _run([idxs[t] for t in PERM], [tabs[t] for t in PERM])
